# baseline (speedup 1.0000x reference)
import functools

import jax
import jax.numpy as jnp
from jax import lax
from jax.experimental import pallas as pl
from jax.experimental.pallas import tpu as pltpu
from jax.experimental.pallas import tpu_sc as plsc

N_TOKENS = 16384
D_MODEL = 2048
N_EXPERTS = 64
T_TILE = 1024
SLAB = 128

CHUNKS = (12288, 4096)

NUM_CORES = 2
NUM_SUBCORES = 16
NUM_WORKERS = NUM_CORES * NUM_SUBCORES
LANES = 16

_STREAM = T_TILE // 2
_SLABS_PER_STREAM = _STREAM // SLAB


def _logits_body(xa_ref, xb_ref, w_ref, out_ref):
    for h, x_ref in enumerate((xa_ref, xb_ref)):
        logits = lax.dot_general(
            x_ref[...], w_ref[...],
            (((1,), (1,)), ((), ())),
            preferred_element_type=jnp.float32,
        )
        lt = logits.T
        for k in range(_SLABS_PER_STREAM):
            out_ref[h * _SLABS_PER_STREAM + k, :, :] = (
                lt[:, k * SLAB:(k + 1) * SLAB])


def _compute_logits_t(x, W, tok_base, ntok):
    base = tok_base // _STREAM
    return pl.pallas_call(
        _logits_body,
        grid=(ntok // T_TILE,),
        in_specs=[
            pl.BlockSpec((_STREAM, D_MODEL), lambda i: (base + 2 * i, 0)),
            pl.BlockSpec((_STREAM, D_MODEL), lambda i: (base + 2 * i + 1, 0)),
            pl.BlockSpec((N_EXPERTS, D_MODEL), lambda i: (0, 0)),
        ],
        out_specs=pl.BlockSpec(
            (2 * _SLABS_PER_STREAM, N_EXPERTS, SLAB),
            lambda i: (i, 0, 0)),
        out_shape=jax.ShapeDtypeStruct(
            (ntok // SLAB, N_EXPERTS, SLAB), jnp.float32),
        compiler_params=pltpu.CompilerParams(
            dimension_semantics=("arbitrary",),
        ),
    )(x, x, W)


def _fused_body(xa_ref, xb_ref, w_ref, i1_ref, i2_ref, g1_ref, g2_ref):
    for h, x_ref in enumerate((xa_ref, xb_ref)):
        logits = lax.dot_general(
            x_ref[...], w_ref[...],
            (((1,), (1,)), ((), ())),
            preferred_element_type=jnp.float32,
        )
        lane = lax.broadcasted_iota(jnp.int32, (_STREAM, N_EXPERTS), 1)
        m1 = jnp.max(logits, axis=1)
        i1 = jnp.min(jnp.where(logits == m1[:, None], lane, N_EXPERTS),
                     axis=1)
        masked = jnp.where(lane == i1[:, None], -jnp.inf, logits)
        m2 = jnp.max(masked, axis=1)
        i2 = jnp.min(jnp.where(masked == m2[:, None], lane, N_EXPERTS),
                     axis=1)
        t = jnp.exp(m2 - m1)
        den = 1.0 + t
        sl = pl.ds(h * _STREAM, _STREAM)
        i1_ref[sl] = i1
        i2_ref[sl] = i2
        g1_ref[sl] = 1.0 / den
        g2_ref[sl] = t / den


def _fused_logits_top2(x, W, tok_base, ntok):
    base = tok_base // _STREAM
    out_i = jax.ShapeDtypeStruct((ntok,), jnp.int32)
    out_f = jax.ShapeDtypeStruct((ntok,), jnp.float32)
    return pl.pallas_call(
        _fused_body,
        grid=(ntok // T_TILE,),
        in_specs=[
            pl.BlockSpec((_STREAM, D_MODEL), lambda i: (base + 2 * i, 0)),
            pl.BlockSpec((_STREAM, D_MODEL), lambda i: (base + 2 * i + 1, 0)),
            pl.BlockSpec((N_EXPERTS, D_MODEL), lambda i: (0, 0)),
        ],
        out_specs=[pl.BlockSpec((T_TILE,), lambda i: (i,))] * 4,
        out_shape=[out_i, out_i, out_f, out_f],
        compiler_params=pltpu.CompilerParams(
            dimension_semantics=("arbitrary",),
        ),
    )(x, x, W)


_SC_MESH = plsc.VectorSubcoreMesh(core_axis_name="c", subcore_axis_name="s")
_GROUPS_PER_SLAB = SLAB // LANES


@functools.lru_cache(maxsize=None)
def _make_sc_top2(ntok):
    chunk = ntok // NUM_WORKERS
    nslab = chunk // SLAB
    groups = chunk // LANES

    @functools.partial(
        pl.kernel,
        out_type=[
            jax.ShapeDtypeStruct((ntok,), jnp.int32),
            jax.ShapeDtypeStruct((ntok,), jnp.int32),
            jax.ShapeDtypeStruct((ntok,), jnp.float32),
            jax.ShapeDtypeStruct((ntok,), jnp.float32),
        ],
        mesh=_SC_MESH,
        scratch_types=[
            pltpu.VMEM((nslab, N_EXPERTS, SLAB), jnp.float32),
            pltpu.VMEM((chunk,), jnp.int32),
            pltpu.VMEM((chunk,), jnp.int32),
            pltpu.VMEM((chunk,), jnp.float32),
            pltpu.VMEM((chunk,), jnp.float32),
        ],
    )
    def _sc_top2(lg_hbm, i1_hbm, i2_hbm, g1_hbm, g2_hbm,
                 lg_v, i1_v, i2_v, g1_v, g2_v):
        wid = lax.axis_index("s") * NUM_CORES + lax.axis_index("c")
        base = wid * chunk
        pltpu.sync_copy(lg_hbm.at[pl.ds(wid * nslab, nslab)], lg_v)

        UNROLL = 4

        def top2_one_group(s, sl):
            m1 = lg_v[s, 0, sl]
            i1 = jnp.zeros((LANES,), jnp.int32)
            m2 = jnp.full((LANES,), -jnp.inf, jnp.float32)
            i2 = jnp.zeros((LANES,), jnp.int32)
            for e in range(1, N_EXPERTS):
                e_vec = jnp.full((LANES,), e, jnp.int32)
                v = lg_v[s, e, sl]
                gt1 = v > m1
                gt2 = v > m2
                i2 = jnp.where(gt1, i1, jnp.where(gt2, e_vec, i2))
                m2 = jnp.maximum(m2, jnp.minimum(v, m1))
                i1 = jnp.where(gt1, e_vec, i1)
                m1 = jnp.maximum(m1, v)
            return m1, i1, m2, i2

        @plsc.parallel_loop(0, groups // UNROLL)
        def group(j):
            for u in range(UNROLL):
                g = j * UNROLL + u
                s = g // _GROUPS_PER_SLAB
                sl = pl.ds((g % _GROUPS_PER_SLAB) * LANES, LANES)
                m1, i1, m2, i2 = top2_one_group(s, sl)
                t = jnp.exp(m2 - m1)
                den = 1.0 + t
                osl = pl.ds(g * LANES, LANES)
                i1_v[osl] = i1
                i2_v[osl] = i2
                g1_v[osl] = 1.0 / den
                g2_v[osl] = t / den
        pltpu.sync_copy(i1_v, i1_hbm.at[pl.ds(base, chunk)])
        pltpu.sync_copy(i2_v, i2_hbm.at[pl.ds(base, chunk)])
        pltpu.sync_copy(g1_v, g1_hbm.at[pl.ds(base, chunk)])
        pltpu.sync_copy(g2_v, g2_hbm.at[pl.ds(base, chunk)])

    return _sc_top2


@jax.jit
def kernel(x, W):
    parts = []
    tok_base = 0
    for ntok in CHUNKS[:-1]:
        lg = _compute_logits_t(x, W, tok_base, ntok)
        parts.append(_make_sc_top2(ntok)(lg))
        tok_base += ntok
    parts.append(_fused_logits_top2(x, W, tok_base, CHUNKS[-1]))
    i1 = jnp.concatenate([p[0] for p in parts])
    i2 = jnp.concatenate([p[1] for p in parts])
    g1 = jnp.concatenate([p[2] for p in parts])
    g2 = jnp.concatenate([p[3] for p in parts])
    idx = jnp.stack([i1, i2], axis=-1)
    gates = jnp.stack([g1, g2], axis=-1)
    return (idx, gates)


# speedup vs baseline: 1.0351x; 1.0351x over previous
import functools

import jax
import jax.numpy as jnp
from jax import lax
from jax.experimental import pallas as pl
from jax.experimental.pallas import tpu as pltpu
from jax.experimental.pallas import tpu_sc as plsc

N_TOKENS = 16384
D_MODEL = 2048
N_EXPERTS = 64
T_TILE = 2048
SLAB = 128

CHUNKS = (16384,)

NUM_CORES = 2
NUM_SUBCORES = 16
NUM_WORKERS = NUM_CORES * NUM_SUBCORES
LANES = 16

_STREAM = T_TILE // 2
_SLABS_PER_STREAM = _STREAM // SLAB


def _logits_body(xa_ref, xb_ref, w_ref, out_ref):
    for h, x_ref in enumerate((xa_ref, xb_ref)):
        logits = lax.dot_general(
            x_ref[...], w_ref[...],
            (((1,), (1,)), ((), ())),
            preferred_element_type=jnp.float32,
        )
        lt = logits.T
        for k in range(_SLABS_PER_STREAM):
            out_ref[h * _SLABS_PER_STREAM + k, :, :] = (
                lt[:, k * SLAB:(k + 1) * SLAB])


def _compute_logits_t(x, W, tok_base, ntok):
    base = tok_base // _STREAM
    return pl.pallas_call(
        _logits_body,
        grid=(ntok // T_TILE,),
        in_specs=[
            pl.BlockSpec((_STREAM, D_MODEL), lambda i: (base + 2 * i, 0)),
            pl.BlockSpec((_STREAM, D_MODEL), lambda i: (base + 2 * i + 1, 0)),
            pl.BlockSpec((N_EXPERTS, D_MODEL), lambda i: (0, 0)),
        ],
        out_specs=pl.BlockSpec(
            (2 * _SLABS_PER_STREAM, N_EXPERTS, SLAB),
            lambda i: (i, 0, 0)),
        out_shape=jax.ShapeDtypeStruct(
            (ntok // SLAB, N_EXPERTS, SLAB), jnp.float32),
        compiler_params=pltpu.CompilerParams(
            dimension_semantics=("arbitrary",),
        ),
    )(x, x, W)


def _fused_body(xa_ref, xb_ref, w_ref, i1_ref, i2_ref, g1_ref, g2_ref):
    for h, x_ref in enumerate((xa_ref, xb_ref)):
        logits = lax.dot_general(
            x_ref[...], w_ref[...],
            (((1,), (1,)), ((), ())),
            preferred_element_type=jnp.float32,
        )
        lane = lax.broadcasted_iota(jnp.int32, (_STREAM, N_EXPERTS), 1)
        m1 = jnp.max(logits, axis=1)
        i1 = jnp.min(jnp.where(logits == m1[:, None], lane, N_EXPERTS),
                     axis=1)
        masked = jnp.where(lane == i1[:, None], -jnp.inf, logits)
        m2 = jnp.max(masked, axis=1)
        i2 = jnp.min(jnp.where(masked == m2[:, None], lane, N_EXPERTS),
                     axis=1)
        t = jnp.exp(m2 - m1)
        den = 1.0 + t
        sl = pl.ds(h * _STREAM, _STREAM)
        i1_ref[sl] = i1
        i2_ref[sl] = i2
        g1_ref[sl] = 1.0 / den
        g2_ref[sl] = t / den


def _fused_logits_top2(x, W, tok_base, ntok):
    base = tok_base // _STREAM
    out_i = jax.ShapeDtypeStruct((ntok,), jnp.int32)
    out_f = jax.ShapeDtypeStruct((ntok,), jnp.float32)
    return pl.pallas_call(
        _fused_body,
        grid=(ntok // T_TILE,),
        in_specs=[
            pl.BlockSpec((_STREAM, D_MODEL), lambda i: (base + 2 * i, 0)),
            pl.BlockSpec((_STREAM, D_MODEL), lambda i: (base + 2 * i + 1, 0)),
            pl.BlockSpec((N_EXPERTS, D_MODEL), lambda i: (0, 0)),
        ],
        out_specs=[pl.BlockSpec((T_TILE,), lambda i: (i,))] * 4,
        out_shape=[out_i, out_i, out_f, out_f],
        compiler_params=pltpu.CompilerParams(
            dimension_semantics=("arbitrary",),
        ),
    )(x, x, W)


_SC_MESH = plsc.VectorSubcoreMesh(core_axis_name="c", subcore_axis_name="s")
_GROUPS_PER_SLAB = SLAB // LANES


@functools.lru_cache(maxsize=None)
def _make_sc_top2(ntok):
    chunk = ntok // NUM_WORKERS
    nslab = chunk // SLAB
    groups = chunk // LANES

    @functools.partial(
        pl.kernel,
        out_type=[
            jax.ShapeDtypeStruct((ntok,), jnp.int32),
            jax.ShapeDtypeStruct((ntok,), jnp.int32),
            jax.ShapeDtypeStruct((ntok,), jnp.float32),
            jax.ShapeDtypeStruct((ntok,), jnp.float32),
        ],
        mesh=_SC_MESH,
        scratch_types=[
            pltpu.VMEM((nslab, N_EXPERTS, SLAB), jnp.float32),
            pltpu.VMEM((chunk,), jnp.int32),
            pltpu.VMEM((chunk,), jnp.int32),
            pltpu.VMEM((chunk,), jnp.float32),
            pltpu.VMEM((chunk,), jnp.float32),
        ],
    )
    def _sc_top2(lg_hbm, i1_hbm, i2_hbm, g1_hbm, g2_hbm,
                 lg_v, i1_v, i2_v, g1_v, g2_v):
        wid = lax.axis_index("s") * NUM_CORES + lax.axis_index("c")
        base = wid * chunk
        pltpu.sync_copy(lg_hbm.at[pl.ds(wid * nslab, nslab)], lg_v)

        UNROLL = 4

        def top2_one_group(s, sl):
            m1 = lg_v[s, 0, sl]
            i1 = jnp.zeros((LANES,), jnp.int32)
            m2 = jnp.full((LANES,), -jnp.inf, jnp.float32)
            i2 = jnp.zeros((LANES,), jnp.int32)
            for e in range(1, N_EXPERTS):
                e_vec = jnp.full((LANES,), e, jnp.int32)
                v = lg_v[s, e, sl]
                gt1 = v > m1
                gt2 = v > m2
                i2 = jnp.where(gt1, i1, jnp.where(gt2, e_vec, i2))
                m2 = jnp.maximum(m2, jnp.minimum(v, m1))
                i1 = jnp.where(gt1, e_vec, i1)
                m1 = jnp.maximum(m1, v)
            return m1, i1, m2, i2

        @plsc.parallel_loop(0, groups // UNROLL)
        def group(j):
            for u in range(UNROLL):
                g = j * UNROLL + u
                s = g // _GROUPS_PER_SLAB
                sl = pl.ds((g % _GROUPS_PER_SLAB) * LANES, LANES)
                m1, i1, m2, i2 = top2_one_group(s, sl)
                t = jnp.exp(m2 - m1)
                den = 1.0 + t
                osl = pl.ds(g * LANES, LANES)
                i1_v[osl] = i1
                i2_v[osl] = i2
                g1_v[osl] = 1.0 / den
                g2_v[osl] = t / den
        pltpu.sync_copy(i1_v, i1_hbm.at[pl.ds(base, chunk)])
        pltpu.sync_copy(i2_v, i2_hbm.at[pl.ds(base, chunk)])
        pltpu.sync_copy(g1_v, g1_hbm.at[pl.ds(base, chunk)])
        pltpu.sync_copy(g2_v, g2_hbm.at[pl.ds(base, chunk)])

    return _sc_top2


@jax.jit
def kernel(x, W):
    parts = []
    tok_base = 0
    for ntok in CHUNKS:
        lg = _compute_logits_t(x, W, tok_base, ntok)
        parts.append(_make_sc_top2(ntok)(lg))
        tok_base += ntok
    i1 = jnp.concatenate([p[0] for p in parts])
    i2 = jnp.concatenate([p[1] for p in parts])
    g1 = jnp.concatenate([p[2] for p in parts])
    g2 = jnp.concatenate([p[3] for p in parts])
    idx = jnp.stack([i1, i2], axis=-1)
    gates = jnp.stack([g1, g2], axis=-1)
    return (idx, gates)


# speedup vs baseline: 1.0591x; 1.0232x over previous
import functools

import jax
import jax.numpy as jnp
from jax import lax
from jax.experimental import pallas as pl
from jax.experimental.pallas import tpu as pltpu
from jax.experimental.pallas import tpu_sc as plsc

N_TOKENS = 16384
D_MODEL = 2048
N_EXPERTS = 64
T_TILE = 1024
SLAB = 128

CHUNKS = (16384,)

NUM_CORES = 2
NUM_SUBCORES = 16
NUM_WORKERS = NUM_CORES * NUM_SUBCORES
LANES = 16

_STREAM = T_TILE // 2
_SLABS_PER_STREAM = _STREAM // SLAB


def _logits_body(xa_ref, xb_ref, w_ref, out_ref):
    for h, x_ref in enumerate((xa_ref, xb_ref)):
        logits = lax.dot_general(
            x_ref[...], w_ref[...],
            (((1,), (1,)), ((), ())),
            preferred_element_type=jnp.float32,
        )
        lt = logits.T
        for k in range(_SLABS_PER_STREAM):
            out_ref[h * _SLABS_PER_STREAM + k, :, :] = (
                lt[:, k * SLAB:(k + 1) * SLAB])


def _compute_logits_t(x, W, tok_base, ntok):
    base = tok_base // _STREAM
    return pl.pallas_call(
        _logits_body,
        grid=(ntok // T_TILE,),
        in_specs=[
            pl.BlockSpec((_STREAM, D_MODEL), lambda i: (base + 2 * i, 0)),
            pl.BlockSpec((_STREAM, D_MODEL), lambda i: (base + 2 * i + 1, 0)),
            pl.BlockSpec((N_EXPERTS, D_MODEL), lambda i: (0, 0)),
        ],
        out_specs=pl.BlockSpec(
            (2 * _SLABS_PER_STREAM, N_EXPERTS, SLAB),
            lambda i: (i, 0, 0)),
        out_shape=jax.ShapeDtypeStruct(
            (ntok // SLAB, N_EXPERTS, SLAB), jnp.float32),
        compiler_params=pltpu.CompilerParams(
            dimension_semantics=("arbitrary",),
        ),
    )(x, x, W)


def _fused_body(xa_ref, xb_ref, w_ref, i1_ref, i2_ref, g1_ref, g2_ref):
    for h, x_ref in enumerate((xa_ref, xb_ref)):
        logits = lax.dot_general(
            x_ref[...], w_ref[...],
            (((1,), (1,)), ((), ())),
            preferred_element_type=jnp.float32,
        )
        lane = lax.broadcasted_iota(jnp.int32, (_STREAM, N_EXPERTS), 1)
        m1 = jnp.max(logits, axis=1)
        i1 = jnp.min(jnp.where(logits == m1[:, None], lane, N_EXPERTS),
                     axis=1)
        masked = jnp.where(lane == i1[:, None], -jnp.inf, logits)
        m2 = jnp.max(masked, axis=1)
        i2 = jnp.min(jnp.where(masked == m2[:, None], lane, N_EXPERTS),
                     axis=1)
        t = jnp.exp(m2 - m1)
        den = 1.0 + t
        sl = pl.ds(h * _STREAM, _STREAM)
        i1_ref[sl] = i1
        i2_ref[sl] = i2
        g1_ref[sl] = 1.0 / den
        g2_ref[sl] = t / den


def _fused_logits_top2(x, W, tok_base, ntok):
    base = tok_base // _STREAM
    out_i = jax.ShapeDtypeStruct((ntok,), jnp.int32)
    out_f = jax.ShapeDtypeStruct((ntok,), jnp.float32)
    return pl.pallas_call(
        _fused_body,
        grid=(ntok // T_TILE,),
        in_specs=[
            pl.BlockSpec((_STREAM, D_MODEL), lambda i: (base + 2 * i, 0)),
            pl.BlockSpec((_STREAM, D_MODEL), lambda i: (base + 2 * i + 1, 0)),
            pl.BlockSpec((N_EXPERTS, D_MODEL), lambda i: (0, 0)),
        ],
        out_specs=[pl.BlockSpec((T_TILE,), lambda i: (i,))] * 4,
        out_shape=[out_i, out_i, out_f, out_f],
        compiler_params=pltpu.CompilerParams(
            dimension_semantics=("arbitrary",),
        ),
    )(x, x, W)


_SC_MESH = plsc.VectorSubcoreMesh(core_axis_name="c", subcore_axis_name="s")
_GROUPS_PER_SLAB = SLAB // LANES


@functools.lru_cache(maxsize=None)
def _make_sc_top2(ntok):
    chunk = ntok // NUM_WORKERS
    nslab = chunk // SLAB
    groups = chunk // LANES

    @functools.partial(
        pl.kernel,
        out_type=[
            jax.ShapeDtypeStruct((ntok,), jnp.int32),
            jax.ShapeDtypeStruct((ntok,), jnp.int32),
            jax.ShapeDtypeStruct((ntok,), jnp.float32),
            jax.ShapeDtypeStruct((ntok,), jnp.float32),
        ],
        mesh=_SC_MESH,
        scratch_types=[
            pltpu.VMEM((nslab, N_EXPERTS, SLAB), jnp.float32),
            pltpu.VMEM((chunk,), jnp.int32),
            pltpu.VMEM((chunk,), jnp.int32),
            pltpu.VMEM((chunk,), jnp.float32),
            pltpu.VMEM((chunk,), jnp.float32),
        ],
    )
    def _sc_top2(lg_hbm, i1_hbm, i2_hbm, g1_hbm, g2_hbm,
                 lg_v, i1_v, i2_v, g1_v, g2_v):
        wid = lax.axis_index("s") * NUM_CORES + lax.axis_index("c")
        base = wid * chunk
        pltpu.sync_copy(lg_hbm.at[pl.ds(wid * nslab, nslab)], lg_v)

        UNROLL = 4

        def top2_one_group(s, sl):
            m1 = lg_v[s, 0, sl]
            i1 = jnp.zeros((LANES,), jnp.int32)
            m2 = jnp.full((LANES,), -jnp.inf, jnp.float32)
            i2 = jnp.zeros((LANES,), jnp.int32)
            for e in range(1, N_EXPERTS):
                e_vec = jnp.full((LANES,), e, jnp.int32)
                v = lg_v[s, e, sl]
                gt1 = v > m1
                gt2 = v > m2
                i2 = jnp.where(gt1, i1, jnp.where(gt2, e_vec, i2))
                m2 = jnp.maximum(m2, jnp.minimum(v, m1))
                i1 = jnp.where(gt1, e_vec, i1)
                m1 = jnp.maximum(m1, v)
            return m1, i1, m2, i2

        @plsc.parallel_loop(0, groups // UNROLL)
        def group(j):
            for u in range(UNROLL):
                g = j * UNROLL + u
                s = g // _GROUPS_PER_SLAB
                sl = pl.ds((g % _GROUPS_PER_SLAB) * LANES, LANES)
                m1, i1, m2, i2 = top2_one_group(s, sl)
                t = jnp.exp(m2 - m1)
                den = 1.0 + t
                osl = pl.ds(g * LANES, LANES)
                i1_v[osl] = i1
                i2_v[osl] = i2
                g1_v[osl] = 1.0 / den
                g2_v[osl] = t / den
        pltpu.sync_copy(i1_v, i1_hbm.at[pl.ds(base, chunk)])
        pltpu.sync_copy(i2_v, i2_hbm.at[pl.ds(base, chunk)])
        pltpu.sync_copy(g1_v, g1_hbm.at[pl.ds(base, chunk)])
        pltpu.sync_copy(g2_v, g2_hbm.at[pl.ds(base, chunk)])

    return _sc_top2


@jax.jit
def kernel(x, W):
    parts = []
    tok_base = 0
    for ntok in CHUNKS:
        lg = _compute_logits_t(x, W, tok_base, ntok)
        parts.append(_make_sc_top2(ntok)(lg))
        tok_base += ntok
    i1 = jnp.concatenate([p[0] for p in parts])
    i2 = jnp.concatenate([p[1] for p in parts])
    g1 = jnp.concatenate([p[2] for p in parts])
    g2 = jnp.concatenate([p[3] for p in parts])
    idx = jnp.stack([i1, i2], axis=-1)
    gates = jnp.stack([g1, g2], axis=-1)
    return (idx, gates)
